# initial kernel scaffold (unmeasured)
import jax
import jax.numpy as jnp
from jax import lax
from jax.experimental import pallas as pl
from jax.experimental.pallas import tpu as pltpu

_BLK = 512


def kernel(partial, resid, gamma):
    m, d = resid.shape
    p = partial[0].astype(jnp.bfloat16)
    r = resid.astype(jnp.bfloat16)

    def body(p_ref, r_ref, g_ref, out_ref, send_sem, recv_sem):
        my_x = lax.axis_index("x")
        my_y = lax.axis_index("y")
        my_z = lax.axis_index("z")
        partner = (1 - my_x, my_y, my_z)

        barrier = pltpu.get_barrier_semaphore()
        pl.semaphore_signal(
            barrier, inc=1, device_id=partner,
            device_id_type=pl.DeviceIdType.MESH,
        )
        pl.semaphore_wait(barrier, 1)

        rdma = pltpu.make_async_remote_copy(
            src_ref=p_ref,
            dst_ref=out_ref,
            send_sem=send_sem,
            recv_sem=recv_sem,
            device_id=partner,
            device_id_type=pl.DeviceIdType.MESH,
        )
        rdma.start()
        rdma.wait()

        g = g_ref[:].astype(jnp.float32)[None, :]
        for i in range(m // _BLK):
            rows = pl.ds(i * _BLK, _BLK)
            y = (
                out_ref[rows, :].astype(jnp.float32)
                + p_ref[rows, :].astype(jnp.float32)
                + r_ref[rows, :].astype(jnp.float32)
            )
            ms = jnp.mean(y * y, axis=-1, keepdims=True)
            out_ref[rows, :] = (y * lax.rsqrt(ms + 1e-6) * g).astype(
                jnp.bfloat16
            )

    return pl.pallas_call(
        body,
        out_shape=jax.ShapeDtypeStruct((m, d), jnp.bfloat16),
        in_specs=[
            pl.BlockSpec(memory_space=pltpu.VMEM),
            pl.BlockSpec(memory_space=pltpu.VMEM),
            pl.BlockSpec(memory_space=pltpu.VMEM),
        ],
        out_specs=pl.BlockSpec(memory_space=pltpu.VMEM),
        scratch_shapes=[
            pltpu.SemaphoreType.DMA,
            pltpu.SemaphoreType.DMA,
        ],
        compiler_params=pltpu.CompilerParams(collective_id=0),
    )(p, r, gamma)


# baseline (device time: 464041 ns/iter reference)
import jax
import jax.numpy as jnp
from jax import lax
from jax.experimental import pallas as pl
from jax.experimental.pallas import tpu as pltpu

_K = 8
_CHUNK = 512
_TILE = 256


def kernel(partial, resid, gamma):
    m, d = resid.shape
    p = partial[0].astype(jnp.bfloat16)
    r = resid.astype(jnp.bfloat16)
    g2 = gamma.reshape(1, -1)

    def body(
        p_hbm, r_hbm, g_ref, out_hbm,
        pp_buf, p_buf, r_buf, o_buf,
        send_sems, recv_sems, in_sems, out_sems,
    ):
        my_x = lax.axis_index("x")
        my_y = lax.axis_index("y")
        my_z = lax.axis_index("z")
        partner = (1 - my_x, my_y, my_z)

        barrier = pltpu.get_barrier_semaphore()
        pl.semaphore_signal(
            barrier, inc=1, device_id=partner,
            device_id_type=pl.DeviceIdType.MESH,
        )
        pl.semaphore_wait(barrier, 1)

        def rows(i):
            return pl.ds(i * _CHUNK, _CHUNK)

        rdmas = []
        for i in range(_K):
            rdma = pltpu.make_async_remote_copy(
                src_ref=p_hbm.at[rows(i), :],
                dst_ref=out_hbm.at[rows(i), :],
                send_sem=send_sems.at[i],
                recv_sem=recv_sems.at[i],
                device_id=partner,
                device_id_type=pl.DeviceIdType.MESH,
            )
            rdma.start()
            rdmas.append(rdma)

        def in_copies(i):
            slot = i % 2
            return [
                pltpu.make_async_copy(
                    out_hbm.at[rows(i), :], pp_buf.at[slot], in_sems.at[slot, 0]
                ),
                pltpu.make_async_copy(
                    p_hbm.at[rows(i), :], p_buf.at[slot], in_sems.at[slot, 1]
                ),
                pltpu.make_async_copy(
                    r_hbm.at[rows(i), :], r_buf.at[slot], in_sems.at[slot, 2]
                ),
            ]

        def out_copy(i):
            slot = i % 2
            return pltpu.make_async_copy(
                o_buf.at[slot], out_hbm.at[rows(i), :], out_sems.at[slot]
            )

        def start_fetch(i):
            rdmas[i].wait_recv()
            for cp in in_copies(i):
                cp.start()

        g = g_ref[:, :].astype(jnp.float32)

        start_fetch(0)
        for i in range(_K):
            slot = i % 2
            if i + 1 < _K:
                start_fetch(i + 1)
            for cp in in_copies(i):
                cp.wait()
            if i >= 2:
                out_copy(i - 2).wait()
            for t in range(_CHUNK // _TILE):
                tr = pl.ds(t * _TILE, _TILE)
                y = (
                    pp_buf[slot, tr, :].astype(jnp.float32)
                    + p_buf[slot, tr, :].astype(jnp.float32)
                    + r_buf[slot, tr, :].astype(jnp.float32)
                )
                ms = jnp.mean(y * y, axis=-1, keepdims=True)
                o_buf[slot, tr, :] = (y * lax.rsqrt(ms + 1e-6) * g).astype(
                    jnp.bfloat16
                )
            out_copy(i).start()

        for i in range(max(0, _K - 2), _K):
            out_copy(i).wait()
        for i in range(_K):
            rdmas[i].wait_send()

    return pl.pallas_call(
        body,
        out_shape=jax.ShapeDtypeStruct((m, d), jnp.bfloat16),
        in_specs=[
            pl.BlockSpec(memory_space=pl.ANY),
            pl.BlockSpec(memory_space=pl.ANY),
            pl.BlockSpec(memory_space=pltpu.VMEM),
        ],
        out_specs=pl.BlockSpec(memory_space=pl.ANY),
        scratch_shapes=[
            pltpu.VMEM((2, _CHUNK, d), jnp.bfloat16),
            pltpu.VMEM((2, _CHUNK, d), jnp.bfloat16),
            pltpu.VMEM((2, _CHUNK, d), jnp.bfloat16),
            pltpu.VMEM((2, _CHUNK, d), jnp.bfloat16),
            pltpu.SemaphoreType.DMA((_K,)),
            pltpu.SemaphoreType.DMA((_K,)),
            pltpu.SemaphoreType.DMA((2, 3)),
            pltpu.SemaphoreType.DMA((2,)),
        ],
        compiler_params=pltpu.CompilerParams(
            collective_id=0, vmem_limit_bytes=60 * 1024 * 1024
        ),
    )(p, r, g2)


# device time: 312287 ns/iter; 1.4859x vs baseline; 1.4859x over previous
import jax
import jax.numpy as jnp
from jax import lax
from jax.experimental import pallas as pl
from jax.experimental.pallas import tpu as pltpu

_S = 256
_C = 2048
_TILE = 128


def kernel(partial, resid, gamma):
    m, d = resid.shape
    p = partial[0].astype(jnp.bfloat16)
    r = resid.astype(jnp.bfloat16)
    g2 = gamma.reshape(1, -1)

    def body(
        p_hbm, r_hbm, g_ref, out_ref,
        pst, ppr, rst,
        loc_sems, x_send, x_recv,
        p1a_send, p1a_recv, p1b_send, p1b_recv,
        p2a_send, p2a_recv, p2b_send, p2b_recv,
    ):
        X = lax.axis_index("x")
        Y = lax.axis_index("y")
        Z = lax.axis_index("z")
        s = 4 * Y + Z
        r0 = s * _S

        cp_p = pltpu.make_async_copy(
            p_hbm.at[pl.ds(r0, _S), :], pst, loc_sems.at[0]
        )
        cp_r = pltpu.make_async_copy(
            r_hbm.at[pl.ds(r0, _S), :], rst, loc_sems.at[1]
        )
        cp_p.start()
        cp_r.start()

        barrier = pltpu.get_barrier_semaphore()

        def sig(dev):
            pl.semaphore_signal(
                barrier, inc=1, device_id=dev,
                device_id_type=pl.DeviceIdType.MESH,
            )

        sig((1 - X, Y, Z))
        pl.when(Y > 0)(lambda: sig((X, Y - 1, Z)))
        pl.when(Y < 3)(lambda: sig((X, Y + 1, Z)))
        pl.when(Z > 0)(lambda: sig((X, Y, Z - 1)))
        pl.when(Z < 3)(lambda: sig((X, Y, Z + 1)))
        pl.semaphore_wait(barrier, 1)
        pl.when(Y > 0)(lambda: pl.semaphore_wait(barrier, 1))
        pl.when(Y < 3)(lambda: pl.semaphore_wait(barrier, 1))
        pl.when(Z > 0)(lambda: pl.semaphore_wait(barrier, 1))
        pl.when(Z < 3)(lambda: pl.semaphore_wait(barrier, 1))

        xr = pltpu.make_async_remote_copy(
            src_ref=p_hbm.at[pl.ds(r0, _S), :],
            dst_ref=ppr,
            send_sem=x_send,
            recv_sem=x_recv,
            device_id=(1 - X, Y, Z),
            device_id_type=pl.DeviceIdType.MESH,
        )
        xr.start()
        xr.wait_recv()
        cp_p.wait()
        cp_r.wait()

        g = g_ref[:, :].astype(jnp.float32)
        for t in range(_S // _TILE):
            tr = pl.ds(t * _TILE, _TILE)
            y32 = (
                pst[tr, :].astype(jnp.float32)
                + ppr[tr, :].astype(jnp.float32)
                + rst[tr, :].astype(jnp.float32)
            )
            ms = jnp.mean(y32 * y32, axis=-1, keepdims=True)
            out_ref[pl.ds(r0 + t * _TILE, _TILE), :] = (
                y32 * lax.rsqrt(ms + 1e-6) * g
            ).astype(jnp.bfloat16)

        started = []

        def mkphase(axis, col0, chunks_fn, send_sems, recv_sems, nchunk):
            pos = Z if axis == "z" else Y

            def dev(off):
                if axis == "z":
                    return (X, Y, Z + off)
                return (X, Y + off, Z)

            def ops(di):
                if di == 0:
                    return (
                        lambda t: (pos >= t) & (pos < 3),
                        lambda t: pos - t,
                        lambda u: pos >= u + 1,
                        lambda u: pos - 1 - u,
                        1,
                    )
                return (
                    lambda t: (pos <= 3 - t) & (pos > 0),
                    lambda t: pos + t,
                    lambda u: pos <= 2 - u,
                    lambda u: pos + 1 + u,
                    -1,
                )

            def mkrdma(di, step, blk, c):
                ro, nr = chunks_fn(jnp.clip(blk, 0, 3))[c]
                idx = (di, step, c) if nchunk > 1 else (di, step)
                return pltpu.make_async_remote_copy(
                    src_ref=out_ref.at[pl.ds(ro, nr), pl.ds(col0, _C)],
                    dst_ref=out_ref.at[pl.ds(ro, nr), pl.ds(col0, _C)],
                    send_sem=send_sems.at[idx],
                    recv_sem=recv_sems.at[idx],
                    device_id=dev(ops(di)[4]),
                    device_id_type=pl.DeviceIdType.MESH,
                )

            def send_step(t):
                for di in range(2):
                    send_ok, send_blk, _, _, _ = ops(di)
                    cond = send_ok(t)
                    for c in range(nchunk):
                        rdma = mkrdma(di, t, send_blk(t), c)
                        pl.when(cond)(lambda rdma=rdma: rdma.start())
                        started.append((cond, rdma))

            def wait_step(u):
                for di in range(2):
                    _, _, recv_ok, recv_blk, _ = ops(di)
                    cond = recv_ok(u)
                    for c in range(nchunk):
                        rdma = mkrdma(di, u, recv_blk(u), c)
                        pl.when(cond)(lambda rdma=rdma: rdma.wait_recv())

            return send_step, wait_step

        p1a = mkphase(
            "z", 0, lambda b: [((4 * Y + b) * _S, _S)], p1a_send, p1a_recv, 1
        )
        p1b = mkphase(
            "y", _C, lambda b: [((4 * b + Z) * _S, _S)], p1b_send, p1b_recv, 1
        )
        p2a = mkphase(
            "y", 0, lambda b: [(b * 4 * _S, 4 * _S)], p2a_send, p2a_recv, 1
        )
        p2b = mkphase(
            "z", _C,
            lambda b: [((4 * yy + b) * _S, _S) for yy in range(4)],
            p2b_send, p2b_recv, 4,
        )

        p1a[0](0)
        p1b[0](0)
        for t in range(1, 4):
            p1a[1](t - 1)
            p1b[1](t - 1)
            if t <= 2:
                p1a[0](t)
                p1b[0](t)

        p2a[0](0)
        p2b[0](0)
        for t in range(1, 4):
            p2a[1](t - 1)
            p2b[1](t - 1)
            if t <= 2:
                p2a[0](t)
                p2b[0](t)

        xr.wait_send()
        for cond, rdma in started:
            pl.when(cond)(lambda rdma=rdma: rdma.wait_send())

    return pl.pallas_call(
        body,
        out_shape=jax.ShapeDtypeStruct((m, d), jnp.bfloat16),
        in_specs=[
            pl.BlockSpec(memory_space=pl.ANY),
            pl.BlockSpec(memory_space=pl.ANY),
            pl.BlockSpec(memory_space=pltpu.VMEM),
        ],
        out_specs=pl.BlockSpec(memory_space=pltpu.VMEM),
        scratch_shapes=[
            pltpu.VMEM((_S, d), jnp.bfloat16),
            pltpu.VMEM((_S, d), jnp.bfloat16),
            pltpu.VMEM((_S, d), jnp.bfloat16),
            pltpu.SemaphoreType.DMA((2,)),
            pltpu.SemaphoreType.DMA,
            pltpu.SemaphoreType.DMA,
            pltpu.SemaphoreType.DMA((2, 3)),
            pltpu.SemaphoreType.DMA((2, 3)),
            pltpu.SemaphoreType.DMA((2, 3)),
            pltpu.SemaphoreType.DMA((2, 3)),
            pltpu.SemaphoreType.DMA((2, 3)),
            pltpu.SemaphoreType.DMA((2, 3)),
            pltpu.SemaphoreType.DMA((2, 3, 4)),
            pltpu.SemaphoreType.DMA((2, 3, 4)),
        ],
        compiler_params=pltpu.CompilerParams(
            collective_id=0, vmem_limit_bytes=60 * 1024 * 1024
        ),
    )(p, r, g2)


# device time: 246029 ns/iter; 1.8861x vs baseline; 1.2693x over previous
import jax
import jax.numpy as jnp
from jax import lax
from jax.experimental import pallas as pl
from jax.experimental.pallas import tpu as pltpu

_S = 256
_C = 2048
_TILE = 128


def kernel(partial, resid, gamma):
    m, d = resid.shape
    p = partial.reshape(m, d)
    g2 = gamma.reshape(1, -1)

    def body(
        p_hbm, r_hbm, g_ref, out_ref,
        pst, pstb, ppr, rst,
        loc_sems, x_send, x_recv,
        p1a_send, p1a_recv, p1b_send, p1b_recv,
        p2a_send, p2a_recv, p2b_send, p2b_recv,
    ):
        X = lax.axis_index("x")
        Y = lax.axis_index("y")
        Z = lax.axis_index("z")
        s = 4 * Y + Z
        r0 = s * _S

        cp_p = pltpu.make_async_copy(
            p_hbm.at[pl.ds(r0, _S), :], pst, loc_sems.at[0]
        )
        cp_r = pltpu.make_async_copy(
            r_hbm.at[pl.ds(r0, _S), :], rst, loc_sems.at[1]
        )
        cp_p.start()
        cp_r.start()

        barrier = pltpu.get_barrier_semaphore()

        def sig(dev):
            pl.semaphore_signal(
                barrier, inc=1, device_id=dev,
                device_id_type=pl.DeviceIdType.MESH,
            )

        sig((1 - X, Y, Z))
        pl.when(Y > 0)(lambda: sig((X, Y - 1, Z)))
        pl.when(Y < 3)(lambda: sig((X, Y + 1, Z)))
        pl.when(Z > 0)(lambda: sig((X, Y, Z - 1)))
        pl.when(Z < 3)(lambda: sig((X, Y, Z + 1)))
        pl.semaphore_wait(barrier, 1)
        pl.when(Y > 0)(lambda: pl.semaphore_wait(barrier, 1))
        pl.when(Y < 3)(lambda: pl.semaphore_wait(barrier, 1))
        pl.when(Z > 0)(lambda: pl.semaphore_wait(barrier, 1))
        pl.when(Z < 3)(lambda: pl.semaphore_wait(barrier, 1))

        cp_p.wait()
        pstb[:, :] = pst[:, :].astype(jnp.bfloat16)
        xr = pltpu.make_async_remote_copy(
            src_ref=pstb,
            dst_ref=ppr,
            send_sem=x_send,
            recv_sem=x_recv,
            device_id=(1 - X, Y, Z),
            device_id_type=pl.DeviceIdType.MESH,
        )
        xr.start()
        xr.wait_recv()
        cp_r.wait()

        g = g_ref[:, :].astype(jnp.float32)
        for t in range(_S // _TILE):
            tr = pl.ds(t * _TILE, _TILE)
            y32 = (
                pst[tr, :]
                + ppr[tr, :].astype(jnp.float32)
                + rst[tr, :]
            )
            ms = jnp.mean(y32 * y32, axis=-1, keepdims=True)
            out_ref[pl.ds(r0 + t * _TILE, _TILE), :] = (
                y32 * lax.rsqrt(ms + 1e-6) * g
            ).astype(jnp.bfloat16)

        started = []

        def mkphase(axis, col0, chunks_fn, send_sems, recv_sems, nchunk):
            pos = Z if axis == "z" else Y

            def dev(off):
                if axis == "z":
                    return (X, Y, Z + off)
                return (X, Y + off, Z)

            def ops(di):
                if di == 0:
                    return (
                        lambda t: (pos >= t) & (pos < 3),
                        lambda t: pos - t,
                        lambda u: pos >= u + 1,
                        lambda u: pos - 1 - u,
                        1,
                    )
                return (
                    lambda t: (pos <= 3 - t) & (pos > 0),
                    lambda t: pos + t,
                    lambda u: pos <= 2 - u,
                    lambda u: pos + 1 + u,
                    -1,
                )

            def mkrdma(di, step, blk, c):
                ro, nr = chunks_fn(jnp.clip(blk, 0, 3))[c]
                idx = (di, step, c) if nchunk > 1 else (di, step)
                return pltpu.make_async_remote_copy(
                    src_ref=out_ref.at[pl.ds(ro, nr), pl.ds(col0, _C)],
                    dst_ref=out_ref.at[pl.ds(ro, nr), pl.ds(col0, _C)],
                    send_sem=send_sems.at[idx],
                    recv_sem=recv_sems.at[idx],
                    device_id=dev(ops(di)[4]),
                    device_id_type=pl.DeviceIdType.MESH,
                )

            def send_step(t):
                for di in range(2):
                    send_ok, send_blk, _, _, _ = ops(di)
                    cond = send_ok(t)
                    for c in range(nchunk):
                        rdma = mkrdma(di, t, send_blk(t), c)
                        pl.when(cond)(lambda rdma=rdma: rdma.start())
                        started.append((cond, rdma))

            def wait_step(u):
                for di in range(2):
                    _, _, recv_ok, recv_blk, _ = ops(di)
                    cond = recv_ok(u)
                    for c in range(nchunk):
                        rdma = mkrdma(di, u, recv_blk(u), c)
                        pl.when(cond)(lambda rdma=rdma: rdma.wait_recv())

            return send_step, wait_step

        p1a = mkphase(
            "z", 0, lambda b: [((4 * Y + b) * _S, _S)], p1a_send, p1a_recv, 1
        )
        p1b = mkphase(
            "y", _C, lambda b: [((4 * b + Z) * _S, _S)], p1b_send, p1b_recv, 1
        )
        p2a = mkphase(
            "y", 0, lambda b: [(b * 4 * _S, 4 * _S)], p2a_send, p2a_recv, 1
        )
        p2b = mkphase(
            "z", _C,
            lambda b: [((4 * yy + b) * _S, _S) for yy in range(4)],
            p2b_send, p2b_recv, 4,
        )

        p1a[0](0)
        p1b[0](0)
        for t in range(1, 4):
            p1a[1](t - 1)
            p1b[1](t - 1)
            if t <= 2:
                p1a[0](t)
                p1b[0](t)

        p2a[0](0)
        p2b[0](0)
        for t in range(1, 4):
            p2a[1](t - 1)
            p2b[1](t - 1)
            if t <= 2:
                p2a[0](t)
                p2b[0](t)

        xr.wait_send()
        for cond, rdma in started:
            pl.when(cond)(lambda rdma=rdma: rdma.wait_send())

    return pl.pallas_call(
        body,
        out_shape=jax.ShapeDtypeStruct((m, d), jnp.bfloat16),
        in_specs=[
            pl.BlockSpec(memory_space=pl.ANY),
            pl.BlockSpec(memory_space=pl.ANY),
            pl.BlockSpec(memory_space=pltpu.VMEM),
        ],
        out_specs=pl.BlockSpec(memory_space=pltpu.VMEM),
        scratch_shapes=[
            pltpu.VMEM((_S, d), jnp.float32),
            pltpu.VMEM((_S, d), jnp.bfloat16),
            pltpu.VMEM((_S, d), jnp.bfloat16),
            pltpu.VMEM((_S, d), jnp.float32),
            pltpu.SemaphoreType.DMA((2,)),
            pltpu.SemaphoreType.DMA,
            pltpu.SemaphoreType.DMA,
            pltpu.SemaphoreType.DMA((2, 3)),
            pltpu.SemaphoreType.DMA((2, 3)),
            pltpu.SemaphoreType.DMA((2, 3)),
            pltpu.SemaphoreType.DMA((2, 3)),
            pltpu.SemaphoreType.DMA((2, 3)),
            pltpu.SemaphoreType.DMA((2, 3)),
            pltpu.SemaphoreType.DMA((2, 3, 4)),
            pltpu.SemaphoreType.DMA((2, 3, 4)),
        ],
        compiler_params=pltpu.CompilerParams(
            collective_id=0, vmem_limit_bytes=60 * 1024 * 1024
        ),
    )(p, resid, g2)
